# Initial kernel scaffold; baseline (speedup 1.0000x reference)
import jax
import jax.numpy as jnp
from jax.experimental import pallas as pl


def kernel(x, poly_index, poly_conn, params):
    raise NotImplementedError("write your pallas kernel here")


# speedup vs baseline: 16.0347x; 16.0347x over previous
import functools

import jax
import jax.numpy as jnp
from jax import lax
from jax.experimental import pallas as pl
from jax.experimental.pallas import tpu as pltpu

N = 10000
E = 320000
HID = 128
H = 8
D = 16
CLAMP = 5.0

BN = 1000
BE = 1280


def _ln_rows(x, g, b):
    m = jnp.mean(x, axis=-1, keepdims=True)
    v = jnp.mean((x - m) ** 2, axis=-1, keepdims=True)
    return (x - m) * lax.rsqrt(v + 1e-05) * g + b


def _qkv_body(x_ref, wq_ref, wk_ref, wv_ref, q_ref, k_ref, v_ref):
    x = x_ref[...]
    q_ref[...] = jnp.dot(x, wq_ref[...], preferred_element_type=jnp.float32)
    k_ref[...] = jnp.dot(x, wk_ref[...], preferred_element_type=jnp.float32)
    v_ref[...] = jnp.dot(x, wv_ref[...], preferred_element_type=jnp.float32)


def _qkv(x, wq, wk, wv):
    full = pl.BlockSpec((HID, HID), lambda i: (0, 0))
    row = pl.BlockSpec((BN, HID), lambda i: (i, 0))
    return pl.pallas_call(
        _qkv_body,
        grid=(N // BN,),
        in_specs=[row, full, full, full],
        out_specs=[row, row, row],
        out_shape=[jax.ShapeDtypeStruct((N, HID), jnp.float32)] * 3,
    )(x, wq, wk, wv)


def _edge_body(pc_ref, qd_ref, ks_ref, vs_ref,
               wew_ref, web_ref, beb_ref, weo_ref, beo_ref,
               s128_ref, t16_ref, woe_ref, boe_ref, g_ref, b_ref,
               e_ref, valv_ref, valc_ref, p16_ref):
    pc = pc_ref[...]
    msg = qd_ref[...] + ks_ref[...]
    ew = jnp.dot(pc, wew_ref[...], preferred_element_type=jnp.float32)
    eb = jnp.dot(pc, web_ref[...], preferred_element_type=jnp.float32) + beb_ref[...]
    c1 = msg * ew
    c2 = jnp.sqrt(jnp.maximum(c1, 0.0)) - jnp.sqrt(jnp.maximum(-c1, 0.0))
    conn = jnp.maximum(c2 + eb, 0.0)
    C = jnp.dot(conn, weo_ref[...], preferred_element_type=jnp.float32) + beo_ref[...]
    score = jnp.clip(jnp.dot(C, s128_ref[...], preferred_element_type=jnp.float32),
                     -CLAMP, CLAMP)
    p128 = jnp.exp(score)
    valv_ref[...] = p128 * vs_ref[...]
    valc_ref[...] = p128 * C
    p16_ref[...] = jnp.dot(p128, t16_ref[...], preferred_element_type=jnp.float32)
    e0 = pc + jnp.dot(C, woe_ref[...], preferred_element_type=jnp.float32) + boe_ref[...]
    e_ref[...] = _ln_rows(e0, g_ref[...], b_ref[...])


def _edge_pipeline(pc, qd, ks, vs, wew, web, beb, weo, beo, s128, t16,
                   woe, boe, g, b):
    row = pl.BlockSpec((BE, HID), lambda i: (i, 0))
    full = pl.BlockSpec((HID, HID), lambda i: (0, 0))
    vec = pl.BlockSpec((1, HID), lambda i: (0, 0))
    t16_spec = pl.BlockSpec((HID, 16), lambda i: (0, 0))
    p16_spec = pl.BlockSpec((BE, 16), lambda i: (i, 0))
    return pl.pallas_call(
        _edge_body,
        grid=(E // BE,),
        in_specs=[row, row, row, row, full, full, vec, full, vec,
                  full, t16_spec, full, vec, vec, vec],
        out_specs=[row, row, row, p16_spec],
        out_shape=[
            jax.ShapeDtypeStruct((E, HID), jnp.float32),
            jax.ShapeDtypeStruct((E, HID), jnp.float32),
            jax.ShapeDtypeStruct((E, HID), jnp.float32),
            jax.ShapeDtypeStruct((E, 16), jnp.float32),
        ],
    )(pc, qd, ks, vs, wew, web, beb, weo, beo, s128, t16, woe, boe, g, b)


def _node_body(x_ref, aggv_ref, rowv_ref, ssum_ref, r2_ref, bd_ref,
               woh_ref, boh_ref, g1_ref, b1_ref, w1_ref, bb1_ref,
               w2_ref, bb2_ref, g2_ref, b2_ref, h_ref):
    ssum = jnp.dot(ssum_ref[...], r2_ref[...], preferred_element_type=jnp.float32)
    inv = 1.0 / (ssum + 1e-16)
    on = aggv_ref[...] * inv + jnp.dot(
        rowv_ref[...] * inv, bd_ref[...], preferred_element_type=jnp.float32)
    h1 = x_ref[...] + jnp.dot(on, woh_ref[...],
                              preferred_element_type=jnp.float32) + boh_ref[...]
    h1 = _ln_rows(h1, g1_ref[...], b1_ref[...])
    t = jnp.maximum(jnp.dot(h1, w1_ref[...],
                            preferred_element_type=jnp.float32) + bb1_ref[...], 0.0)
    h2 = jnp.dot(t, w2_ref[...], preferred_element_type=jnp.float32) + bb2_ref[...]
    h_ref[...] = _ln_rows(h1 + h2, g2_ref[...], b2_ref[...])


def _node_epilogue(x, aggv, rowv, ssum16, r2, bd, woh, boh, g1, b1,
                   w1, bb1, w2, bb2, g2, b2):
    row = pl.BlockSpec((BN, HID), lambda i: (i, 0))
    full = pl.BlockSpec((HID, HID), lambda i: (0, 0))
    vec = pl.BlockSpec((1, HID), lambda i: (0, 0))
    return pl.pallas_call(
        _node_body,
        grid=(N // BN,),
        in_specs=[row, row, row,
                  pl.BlockSpec((BN, 16), lambda i: (i, 0)),
                  pl.BlockSpec((16, HID), lambda i: (0, 0)),
                  full, full, vec, vec, vec,
                  pl.BlockSpec((HID, 2 * HID), lambda i: (0, 0)),
                  pl.BlockSpec((1, 2 * HID), lambda i: (0, 0)),
                  pl.BlockSpec((2 * HID, HID), lambda i: (0, 0)),
                  vec, vec, vec],
        out_specs=row,
        out_shape=jax.ShapeDtypeStruct((N, HID), jnp.float32),
    )(x, aggv, rowv, ssum16, r2, bd, woh, boh, g1, b1, w1, bb1, w2, bb2, g2, b2)


def _gather_qkv(qh, kh, vh, dst, src):
    return qh[dst], kh[src], vh[src]


def _scatter_segments(valv, valc, p16, dst):
    aggv = jax.ops.segment_sum(valv, dst, num_segments=N)
    rowv = jax.ops.segment_sum(valc, dst, num_segments=N)
    ssum16 = jax.ops.segment_sum(p16, dst, num_segments=N)
    return aggv, rowv, ssum16


def kernel(x, poly_index, poly_conn, params):
    p = params
    dst = poly_index[0]
    src = poly_index[1]

    aw = p['Aw'][:, :, 0]
    lane = jnp.arange(HID)
    head = lane // D
    s128 = jax.scipy.linalg.block_diag(
        *[jnp.outer(aw[:, h], jnp.ones((D,), jnp.float32)) for h in range(H)])
    t16 = jnp.zeros((HID, 16), jnp.float32).at[lane, head].set(1.0 / D)
    r2 = jnp.zeros((16, HID), jnp.float32).at[head, lane].set(1.0)
    bd = jax.scipy.linalg.block_diag(*[p['BW'][:, h, :] for h in range(H)])

    def v(a):
        return a.reshape(1, -1)

    qh, kh, vh = _qkv(x, p['WQ'], p['WK'], p['WV'])
    qd, ks, vs = _gather_qkv(qh, kh, vh, dst, src)
    e, valv, valc, p16 = _edge_pipeline(
        poly_conn, qd, ks, vs, p['WEw'], p['WEb'], v(p['bEb']), p['WEo'],
        v(p['bEo']), s128, t16, p['WOe'], v(p['bOe']),
        v(p['ln1e_g']), v(p['ln1e_b']))
    aggv, rowv, ssum16 = _scatter_segments(valv, valc, p16, dst)
    h = _node_epilogue(
        x, aggv, rowv, ssum16, r2, bd, p['WOh'], v(p['bOh']),
        v(p['ln1h_g']), v(p['ln1h_b']), p['W1'], v(p['b1']),
        p['W2'], v(p['b2']), v(p['ln2h_g']), v(p['ln2h_b']))
    return (h, e)


# speedup vs baseline: 40.6613x; 2.5358x over previous
import functools

import jax
import jax.numpy as jnp
from jax import lax
from jax.experimental import pallas as pl
from jax.experimental.pallas import tpu as pltpu

N = 10000
E = 320000
HID = 128
H = 8
D = 16
CLAMP = 5.0

BN = 1000
BE = 1280


def _ln_rows(x, g, b):
    m = jnp.mean(x, axis=-1, keepdims=True)
    v = jnp.mean((x - m) ** 2, axis=-1, keepdims=True)
    return (x - m) * lax.rsqrt(v + 1e-05) * g + b


def _qkv_body(x_ref, wq_ref, wk_ref, wv_ref, q_ref, k_ref, v_ref):
    x = x_ref[...]
    q_ref[...] = jnp.dot(x, wq_ref[...], preferred_element_type=jnp.float32)
    k_ref[...] = jnp.dot(x, wk_ref[...], preferred_element_type=jnp.float32)
    v_ref[...] = jnp.dot(x, wv_ref[...], preferred_element_type=jnp.float32)


def _qkv(x, wq, wk, wv):
    full = pl.BlockSpec((HID, HID), lambda i: (0, 0))
    row = pl.BlockSpec((BN, HID), lambda i: (i, 0))
    return pl.pallas_call(
        _qkv_body,
        grid=(N // BN,),
        in_specs=[row, full, full, full],
        out_specs=[row, row, row],
        out_shape=[jax.ShapeDtypeStruct((N, HID), jnp.float32)] * 3,
    )(x, wq, wk, wv)


def _edge_body(pc_ref, qd_ref, ks_ref, vs_ref,
               wew_ref, web_ref, beb_ref, weo_ref, beo_ref,
               s128_ref, woe_ref, boe_ref, g_ref, b_ref,
               e_ref, valv_ref, valc_ref, p128_ref):
    pc = pc_ref[...]
    msg = qd_ref[...] + ks_ref[...]
    ew = jnp.dot(pc, wew_ref[...], preferred_element_type=jnp.float32)
    eb = jnp.dot(pc, web_ref[...], preferred_element_type=jnp.float32) + beb_ref[...]
    c1 = msg * ew
    c2 = jnp.sqrt(jnp.maximum(c1, 0.0)) - jnp.sqrt(jnp.maximum(-c1, 0.0))
    conn = jnp.maximum(c2 + eb, 0.0)
    C = jnp.dot(conn, weo_ref[...], preferred_element_type=jnp.float32) + beo_ref[...]
    score = jnp.clip(jnp.dot(C, s128_ref[...], preferred_element_type=jnp.float32),
                     -CLAMP, CLAMP)
    p128 = jnp.exp(score)
    valv_ref[...] = p128 * vs_ref[...]
    valc_ref[...] = p128 * C
    p128_ref[...] = p128
    e0 = pc + jnp.dot(C, woe_ref[...], preferred_element_type=jnp.float32) + boe_ref[...]
    e_ref[...] = _ln_rows(e0, g_ref[...], b_ref[...])


def _edge_pipeline(pc, qd, ks, vs, wew, web, beb, weo, beo, s128,
                   woe, boe, g, b):
    row = pl.BlockSpec((BE, HID), lambda i: (i, 0))
    full = pl.BlockSpec((HID, HID), lambda i: (0, 0))
    vec = pl.BlockSpec((1, HID), lambda i: (0, 0))
    return pl.pallas_call(
        _edge_body,
        grid=(E // BE,),
        in_specs=[row, row, row, row, full, full, vec, full, vec,
                  full, full, vec, vec, vec],
        out_specs=[row, row, row, row],
        out_shape=[jax.ShapeDtypeStruct((E, HID), jnp.float32)] * 4,
    )(pc, qd, ks, vs, wew, web, beb, weo, beo, s128, woe, boe, g, b)


def _node_body(x_ref, aggv_ref, rowv_ref, ss0_ref, ss1_ref, bd_ref,
               woh_ref, boh_ref, g1_ref, b1_ref, w1_ref, bb1_ref,
               w2_ref, bb2_ref, g2_ref, b2_ref, h_ref):
    inv = 1.0 / (ss0_ref[...] + ss1_ref[...] + 1e-16)
    on = aggv_ref[...] * inv + jnp.dot(
        rowv_ref[...] * inv, bd_ref[...], preferred_element_type=jnp.float32)
    h1 = x_ref[...] + jnp.dot(on, woh_ref[...],
                              preferred_element_type=jnp.float32) + boh_ref[...]
    h1 = _ln_rows(h1, g1_ref[...], b1_ref[...])
    t = jnp.maximum(jnp.dot(h1, w1_ref[...],
                            preferred_element_type=jnp.float32) + bb1_ref[...], 0.0)
    h2 = jnp.dot(t, w2_ref[...], preferred_element_type=jnp.float32) + bb2_ref[...]
    h_ref[...] = _ln_rows(h1 + h2, g2_ref[...], b2_ref[...])


def _node_epilogue(x, aggv, rowv, ssum0, ssum1, bd, woh, boh, g1, b1,
                   w1, bb1, w2, bb2, g2, b2):
    row = pl.BlockSpec((BN, HID), lambda i: (i, 0))
    full = pl.BlockSpec((HID, HID), lambda i: (0, 0))
    vec = pl.BlockSpec((1, HID), lambda i: (0, 0))
    return pl.pallas_call(
        _node_body,
        grid=(N // BN,),
        in_specs=[row, row, row, row, row, full, full, vec, vec, vec,
                  pl.BlockSpec((HID, 2 * HID), lambda i: (0, 0)),
                  pl.BlockSpec((1, 2 * HID), lambda i: (0, 0)),
                  pl.BlockSpec((2 * HID, HID), lambda i: (0, 0)),
                  vec, vec, vec],
        out_specs=row,
        out_shape=jax.ShapeDtypeStruct((N, HID), jnp.float32),
    )(x, aggv, rowv, ssum0, ssum1, bd, woh, boh, g1, b1, w1, bb1, w2, bb2, g2, b2)


from jax.experimental.pallas import tpu_sc as plsc

_NC, _NS = 2, 16
_NW = _NC * _NS
_CH = 80


def _gather_body(qh, kh, vh, dst, src, qd, ks, vs,
                 dstv, srcv, qbuf, kbuf, vbuf, sem):
    epw = E // _NW
    wid = lax.axis_index("s") * _NC + lax.axis_index("c")
    base = wid * epw

    def step(i, carry):
        off = base + i * _CH
        pltpu.sync_copy(dst.at[pl.ds(off, _CH)], dstv)
        pltpu.sync_copy(src.at[pl.ds(off, _CH)], srcv)
        cq = pltpu.async_copy(qh.at[dstv], qbuf, sem)
        ck = pltpu.async_copy(kh.at[srcv], kbuf, sem)
        cv = pltpu.async_copy(vh.at[srcv], vbuf, sem)
        cq.wait()
        ck.wait()
        cv.wait()
        pltpu.sync_copy(qbuf, qd.at[pl.ds(off, _CH)])
        pltpu.sync_copy(kbuf, ks.at[pl.ds(off, _CH)])
        pltpu.sync_copy(vbuf, vs.at[pl.ds(off, _CH)])
        return carry

    lax.fori_loop(0, epw // _CH, step, 0)


def _gather_qkv(qh, kh, vh, dst, src):
    mesh = plsc.VectorSubcoreMesh(core_axis_name="c", subcore_axis_name="s",
                                  num_cores=_NC, num_subcores=_NS)
    fn = pl.kernel(
        _gather_body,
        out_type=[jax.ShapeDtypeStruct((E, HID), jnp.float32)] * 3,
        mesh=mesh,
        scratch_types=[
            pltpu.VMEM((_CH,), jnp.int32),
            pltpu.VMEM((_CH,), jnp.int32),
            pltpu.VMEM((_CH, HID), jnp.float32),
            pltpu.VMEM((_CH, HID), jnp.float32),
            pltpu.VMEM((_CH, HID), jnp.float32),
            pltpu.SemaphoreType.DMA,
        ],
    )
    return fn(qh, kh, vh, dst, src)


def _scatter_body(valv, valc, p128, dst, zrow128, aggv_out, rowv_out,
                  ss0_out, ss1_out, accum, dstv, vbuf):
    c = lax.axis_index("c")
    s = lax.axis_index("s")
    row0 = s * 624
    tail = (s == _NS - 1)
    pltpu.sync_copy(zrow128, vbuf)

    def zero_own_rows():
        def zstripe(j, carry):
            pltpu.sync_copy(vbuf.at[pl.ds(0, 48)],
                            accum.at[pl.ds(row0 + j * 48, 48)])
            return carry

        lax.fori_loop(0, 13, zstripe, 0)

        @pl.when(tail)
        def _():
            pltpu.sync_copy(vbuf.at[pl.ds(0, 16)], accum.at[pl.ds(9984, 16)])

    def write_own_rows(out):
        def wstripe(j, carry):
            r = row0 + j * 48
            pltpu.sync_copy(accum.at[pl.ds(r, 48)], vbuf.at[pl.ds(0, 48)])
            pltpu.sync_copy(vbuf.at[pl.ds(0, 48)], out.at[pl.ds(r, 48)])
            return carry

        lax.fori_loop(0, 13, wstripe, 0)

        @pl.when(tail)
        def _():
            pltpu.sync_copy(accum.at[pl.ds(9984, 16)], vbuf.at[pl.ds(0, 16)])
            pltpu.sync_copy(vbuf.at[pl.ds(0, 16)], out.at[pl.ds(9984, 16)])

    def scatter_range(val, base, nstep):
        def step(i, carry):
            off = base + i * _CH
            pltpu.sync_copy(dst.at[pl.ds(off, _CH)], dstv)
            pltpu.sync_copy(val.at[pl.ds(off, _CH)], vbuf)
            pltpu.sync_copy(vbuf, accum.at[dstv], add=True)
            return carry

        lax.fori_loop(0, nstep, step, 0)

    zero_own_rows()
    plsc.subcore_barrier()

    @pl.when(c == 0)
    def _():
        scatter_range(valv, s * (E // _NS), (E // _NS) // _CH)

    @pl.when(c == 1)
    def _():
        scatter_range(valc, s * (E // _NS), (E // _NS) // _CH)

    plsc.subcore_barrier()

    @pl.when(c == 0)
    def _():
        write_own_rows(aggv_out)

    @pl.when(c == 1)
    def _():
        write_own_rows(rowv_out)

    pltpu.sync_copy(zrow128, vbuf)
    zero_own_rows()
    plsc.subcore_barrier()
    scatter_range(p128, (c * _NS + s) * (E // _NW), (E // _NW) // _CH)
    plsc.subcore_barrier()

    @pl.when(c == 0)
    def _():
        write_own_rows(ss0_out)

    @pl.when(c == 1)
    def _():
        write_own_rows(ss1_out)


def _scatter_segments(valv, valc, p128, dst):
    mesh = plsc.VectorSubcoreMesh(core_axis_name="c", subcore_axis_name="s",
                                  num_cores=_NC, num_subcores=_NS)
    fn = pl.kernel(
        _scatter_body,
        out_type=[jax.ShapeDtypeStruct((N, HID), jnp.float32)] * 4,
        mesh=mesh,
        scratch_types=[
            pltpu.VMEM_SHARED((N, HID), jnp.float32),
            pltpu.VMEM((_CH,), jnp.int32),
            pltpu.VMEM((_CH, HID), jnp.float32),
        ],
    )
    zrow128 = jnp.zeros((_CH, HID), jnp.float32)
    return fn(valv, valc, p128, dst, zrow128)


def kernel(x, poly_index, poly_conn, params):
    p = params
    dst = poly_index[0]
    src = poly_index[1]

    aw = p['Aw'][:, :, 0]
    lane = jnp.arange(HID)
    head = lane // D
    s128 = jax.scipy.linalg.block_diag(
        *[jnp.outer(aw[:, h], jnp.ones((D,), jnp.float32)) for h in range(H)])
    bd = jax.scipy.linalg.block_diag(*[p['BW'][:, h, :] for h in range(H)])

    def v(a):
        return a.reshape(1, -1)

    qh, kh, vh = _qkv(x, p['WQ'], p['WK'], p['WV'])
    qd, ks, vs = _gather_qkv(qh, kh, vh, dst, src)
    e, valv, valc, p128 = _edge_pipeline(
        poly_conn, qd, ks, vs, p['WEw'], p['WEb'], v(p['bEb']), p['WEo'],
        v(p['bEo']), s128, p['WOe'], v(p['bOe']),
        v(p['ln1e_g']), v(p['ln1e_b']))
    aggv, rowv, ss0, ss1 = _scatter_segments(valv, valc, p128, dst)
    h = _node_epilogue(
        x, aggv, rowv, ss0, ss1, bd, p['WOh'], v(p['bOh']),
        v(p['ln1h_g']), v(p['ln1h_b']), p['W1'], v(p['b1']),
        p['W2'], v(p['b2']), v(p['ln2h_g']), v(p['ln2h_b']))
    return (h, e)


# speedup vs baseline: 48.5619x; 1.1943x over previous
import functools

import jax
import jax.numpy as jnp
from jax import lax
from jax.experimental import pallas as pl
from jax.experimental.pallas import tpu as pltpu

N = 10000
E = 320000
HID = 128
H = 8
D = 16
CLAMP = 5.0

BN = 1000
BE = 1280


def _ln_rows(x, g, b):
    m = jnp.mean(x, axis=-1, keepdims=True)
    v = jnp.mean((x - m) ** 2, axis=-1, keepdims=True)
    return (x - m) * lax.rsqrt(v + 1e-05) * g + b


def _qkv_body(x_ref, wq_ref, wk_ref, wv_ref, q_ref, k_ref, v_ref):
    x = x_ref[...]
    q_ref[...] = jnp.dot(x, wq_ref[...], preferred_element_type=jnp.float32)
    k_ref[...] = jnp.dot(x, wk_ref[...], preferred_element_type=jnp.float32)
    v_ref[...] = jnp.dot(x, wv_ref[...], preferred_element_type=jnp.float32)


def _qkv(x, wq, wk, wv):
    full = pl.BlockSpec((HID, HID), lambda i: (0, 0))
    row = pl.BlockSpec((BN, HID), lambda i: (i, 0))
    return pl.pallas_call(
        _qkv_body,
        grid=(N // BN,),
        in_specs=[row, full, full, full],
        out_specs=[row, row, row],
        out_shape=[jax.ShapeDtypeStruct((N, HID), jnp.float32)] * 3,
    )(x, wq, wk, wv)


def _edge_body(pc_ref, qd_ref, ks_ref, vs_ref,
               wew_ref, web_ref, beb_ref, weo_ref, beo_ref,
               s128_ref, woe_ref, boe_ref, g_ref, b_ref,
               e_ref, valv_ref, valc_ref, p128_ref):
    pc = pc_ref[...]
    msg = qd_ref[...] + ks_ref[...]
    ew = jnp.dot(pc, wew_ref[...], preferred_element_type=jnp.float32)
    eb = jnp.dot(pc, web_ref[...], preferred_element_type=jnp.float32) + beb_ref[...]
    c1 = msg * ew
    c2 = jnp.sqrt(jnp.maximum(c1, 0.0)) - jnp.sqrt(jnp.maximum(-c1, 0.0))
    conn = jnp.maximum(c2 + eb, 0.0)
    C = jnp.dot(conn, weo_ref[...], preferred_element_type=jnp.float32) + beo_ref[...]
    score = jnp.clip(jnp.dot(C, s128_ref[...], preferred_element_type=jnp.float32),
                     -CLAMP, CLAMP)
    p128 = jnp.exp(score)
    valv_ref[...] = p128 * vs_ref[...]
    valc_ref[...] = p128 * C
    p128_ref[...] = p128
    e0 = pc + jnp.dot(C, woe_ref[...], preferred_element_type=jnp.float32) + boe_ref[...]
    e_ref[...] = _ln_rows(e0, g_ref[...], b_ref[...])


def _edge_pipeline(pc, qd, ks, vs, wew, web, beb, weo, beo, s128,
                   woe, boe, g, b):
    row = pl.BlockSpec((BE, HID), lambda i: (i, 0))
    full = pl.BlockSpec((HID, HID), lambda i: (0, 0))
    vec = pl.BlockSpec((1, HID), lambda i: (0, 0))
    return pl.pallas_call(
        _edge_body,
        grid=(E // BE,),
        in_specs=[row, row, row, row, full, full, vec, full, vec,
                  full, full, vec, vec, vec],
        out_specs=[row, row, row, row],
        out_shape=[jax.ShapeDtypeStruct((E, HID), jnp.float32)] * 4,
    )(pc, qd, ks, vs, wew, web, beb, weo, beo, s128, woe, boe, g, b)


def _node_body(x_ref, aggv_ref, rowv_ref, ss0_ref, ss1_ref, bd_ref,
               woh_ref, boh_ref, g1_ref, b1_ref, w1_ref, bb1_ref,
               w2_ref, bb2_ref, g2_ref, b2_ref, h_ref):
    inv = 1.0 / (ss0_ref[...] + ss1_ref[...] + 1e-16)
    on = aggv_ref[...] * inv + jnp.dot(
        rowv_ref[...] * inv, bd_ref[...], preferred_element_type=jnp.float32)
    h1 = x_ref[...] + jnp.dot(on, woh_ref[...],
                              preferred_element_type=jnp.float32) + boh_ref[...]
    h1 = _ln_rows(h1, g1_ref[...], b1_ref[...])
    t = jnp.maximum(jnp.dot(h1, w1_ref[...],
                            preferred_element_type=jnp.float32) + bb1_ref[...], 0.0)
    h2 = jnp.dot(t, w2_ref[...], preferred_element_type=jnp.float32) + bb2_ref[...]
    h_ref[...] = _ln_rows(h1 + h2, g2_ref[...], b2_ref[...])


def _node_epilogue(x, aggv, rowv, ssum0, ssum1, bd, woh, boh, g1, b1,
                   w1, bb1, w2, bb2, g2, b2):
    row = pl.BlockSpec((BN, HID), lambda i: (i, 0))
    full = pl.BlockSpec((HID, HID), lambda i: (0, 0))
    vec = pl.BlockSpec((1, HID), lambda i: (0, 0))
    return pl.pallas_call(
        _node_body,
        grid=(N // BN,),
        in_specs=[row, row, row, row, row, full, full, vec, vec, vec,
                  pl.BlockSpec((HID, 2 * HID), lambda i: (0, 0)),
                  pl.BlockSpec((1, 2 * HID), lambda i: (0, 0)),
                  pl.BlockSpec((2 * HID, HID), lambda i: (0, 0)),
                  vec, vec, vec],
        out_specs=row,
        out_shape=jax.ShapeDtypeStruct((N, HID), jnp.float32),
    )(x, aggv, rowv, ssum0, ssum1, bd, woh, boh, g1, b1, w1, bb1, w2, bb2, g2, b2)


from jax.experimental.pallas import tpu_sc as plsc

_NC, _NS = 2, 16
_NW = _NC * _NS
_CH = 80


def _gather_body(qh, kh, vh, dst, src, qd, ks, vs,
                 dstv, srcv, qbuf, kbuf, vbuf, sem):
    epw = E // _NW
    wid = lax.axis_index("s") * _NC + lax.axis_index("c")
    base = wid * epw

    def step(i, carry):
        off = base + i * _CH
        pltpu.sync_copy(dst.at[pl.ds(off, _CH)], dstv)
        pltpu.sync_copy(src.at[pl.ds(off, _CH)], srcv)
        cq = pltpu.async_copy(qh.at[dstv], qbuf, sem)
        ck = pltpu.async_copy(kh.at[srcv], kbuf, sem)
        cv = pltpu.async_copy(vh.at[srcv], vbuf, sem)
        cq.wait()
        ck.wait()
        cv.wait()
        pltpu.sync_copy(qbuf, qd.at[pl.ds(off, _CH)])
        pltpu.sync_copy(kbuf, ks.at[pl.ds(off, _CH)])
        pltpu.sync_copy(vbuf, vs.at[pl.ds(off, _CH)])
        return carry

    lax.fori_loop(0, epw // _CH, step, 0)


def _gather_qkv(qh, kh, vh, dst, src):
    mesh = plsc.VectorSubcoreMesh(core_axis_name="c", subcore_axis_name="s",
                                  num_cores=_NC, num_subcores=_NS)
    fn = pl.kernel(
        _gather_body,
        out_type=[jax.ShapeDtypeStruct((E, HID), jnp.float32)] * 3,
        mesh=mesh,
        scratch_types=[
            pltpu.VMEM((_CH,), jnp.int32),
            pltpu.VMEM((_CH,), jnp.int32),
            pltpu.VMEM((_CH, HID), jnp.float32),
            pltpu.VMEM((_CH, HID), jnp.float32),
            pltpu.VMEM((_CH, HID), jnp.float32),
            pltpu.SemaphoreType.DMA,
        ],
    )
    return fn(qh, kh, vh, dst, src)


def _scatter_body(valv, valc, p128, dst3b, zrow128, aggv_out, rowv_out,
                  ss0_out, ss1_out, accum, dstv, vbuf0, vbuf1, sem0, sem1):
    c = lax.axis_index("c")
    s = lax.axis_index("s")
    row0 = s * 624
    tail = (s == _NS - 1)

    def zero_own_rows():
        pltpu.sync_copy(zrow128, vbuf0)

        def zstripe(j, carry):
            pltpu.sync_copy(vbuf0.at[pl.ds(0, 48)],
                            accum.at[pl.ds(row0 + j * 48, 48)])
            return carry

        lax.fori_loop(0, 13, zstripe, 0)

        @pl.when(tail)
        def _():
            pltpu.sync_copy(vbuf0.at[pl.ds(0, 16)], accum.at[pl.ds(9984, 16)])

    def write_own_rows(out):
        def wstripe(j, carry):
            r = row0 + j * 48
            pltpu.sync_copy(accum.at[pl.ds(r, 48)], vbuf0.at[pl.ds(0, 48)])
            pltpu.sync_copy(vbuf0.at[pl.ds(0, 48)], out.at[pl.ds(r, 48)])
            return carry

        lax.fori_loop(0, 13, wstripe, 0)

        @pl.when(tail)
        def _():
            pltpu.sync_copy(accum.at[pl.ds(9984, 16)], vbuf0.at[pl.ds(0, 16)])
            pltpu.sync_copy(vbuf0.at[pl.ds(0, 16)], out.at[pl.ds(9984, 16)])

    def scatter_range(val, idxrows, edge0, nstep):
        pltpu.sync_copy(idxrows, dstv.at[pl.ds(0, nstep)])

        def pair(j, carry):
            i0 = 2 * j
            i1 = i0 + 1

            @pl.when(j > 0)
            def _():
                pltpu.make_async_copy(
                    vbuf0, accum.at[dstv.at[i0 - 2]], sem0).wait()

            pltpu.sync_copy(val.at[pl.ds(edge0 + i0 * _CH, _CH)], vbuf0)
            pltpu.async_copy(vbuf0, accum.at[dstv.at[i0]], sem0, add=True)

            @pl.when(j > 0)
            def _():
                pltpu.make_async_copy(
                    vbuf1, accum.at[dstv.at[i1 - 2]], sem1).wait()

            pltpu.sync_copy(val.at[pl.ds(edge0 + i1 * _CH, _CH)], vbuf1)
            pltpu.async_copy(vbuf1, accum.at[dstv.at[i1]], sem1, add=True)
            return carry

        npair = nstep // 2
        lax.fori_loop(0, npair, pair, 0)
        pltpu.make_async_copy(
            vbuf0, accum.at[dstv.at[2 * npair - 2]], sem0).wait()
        pltpu.make_async_copy(
            vbuf1, accum.at[dstv.at[2 * npair - 1]], sem1).wait()
        if nstep % 2:
            i = nstep - 1
            pltpu.sync_copy(val.at[pl.ds(edge0 + i * _CH, _CH)], vbuf0)
            pltpu.sync_copy(vbuf0, accum.at[dstv.at[i]], add=True)

    zero_own_rows()
    plsc.subcore_barrier()
    cpw = (E // _NW) // _CH

    @pl.when(c == 0)
    def _():
        scatter_range(valv, dst3b.at[2 * s], 2 * s * cpw * _CH, cpw)
        scatter_range(valv, dst3b.at[2 * s + 1], (2 * s + 1) * cpw * _CH, cpw)

    @pl.when(c == 1)
    def _():
        scatter_range(valc, dst3b.at[2 * s], 2 * s * cpw * _CH, cpw)
        scatter_range(valc, dst3b.at[2 * s + 1], (2 * s + 1) * cpw * _CH, cpw)

    plsc.subcore_barrier()

    @pl.when(c == 0)
    def _():
        write_own_rows(aggv_out)

    @pl.when(c == 1)
    def _():
        write_own_rows(rowv_out)

    zero_own_rows()
    plsc.subcore_barrier()
    w = c * _NS + s
    scatter_range(p128, dst3b.at[w], w * cpw * _CH, cpw)
    plsc.subcore_barrier()

    @pl.when(c == 0)
    def _():
        write_own_rows(ss0_out)

    @pl.when(c == 1)
    def _():
        write_own_rows(ss1_out)


def _scatter_segments(valv, valc, p128, dst):
    mesh = plsc.VectorSubcoreMesh(core_axis_name="c", subcore_axis_name="s",
                                  num_cores=_NC, num_subcores=_NS)
    fn = pl.kernel(
        _scatter_body,
        out_type=[jax.ShapeDtypeStruct((N, HID), jnp.float32)] * 4,
        mesh=mesh,
        scratch_types=[
            pltpu.VMEM_SHARED((N, HID), jnp.float32),
            pltpu.VMEM((E // _NW // _CH, _CH), jnp.int32),
            pltpu.VMEM((_CH, HID), jnp.float32),
            pltpu.VMEM((_CH, HID), jnp.float32),
            pltpu.SemaphoreType.DMA,
            pltpu.SemaphoreType.DMA,
        ],
    )
    zrow128 = jnp.zeros((_CH, HID), jnp.float32)
    dst3b = dst.reshape(_NW, E // _NW // _CH, _CH)
    return fn(valv, valc, p128, dst3b, zrow128)


def kernel(x, poly_index, poly_conn, params):
    p = params
    dst = poly_index[0]
    src = poly_index[1]

    aw = p['Aw'][:, :, 0]
    lane = jnp.arange(HID)
    head = lane // D
    s128 = jax.scipy.linalg.block_diag(
        *[jnp.outer(aw[:, h], jnp.ones((D,), jnp.float32)) for h in range(H)])
    bd = jax.scipy.linalg.block_diag(*[p['BW'][:, h, :] for h in range(H)])

    def v(a):
        return a.reshape(1, -1)

    qh, kh, vh = _qkv(x, p['WQ'], p['WK'], p['WV'])
    qd, ks, vs = _gather_qkv(qh, kh, vh, dst, src)
    e, valv, valc, p128 = _edge_pipeline(
        poly_conn, qd, ks, vs, p['WEw'], p['WEb'], v(p['bEb']), p['WEo'],
        v(p['bEo']), s128, p['WOe'], v(p['bOe']),
        v(p['ln1e_g']), v(p['ln1e_b']))
    aggv, rowv, ss0, ss1 = _scatter_segments(valv, valc, p128, dst)
    h = _node_epilogue(
        x, aggv, rowv, ss0, ss1, bd, p['WOh'], v(p['bOh']),
        v(p['ln1h_g']), v(p['ln1h_b']), p['W1'], v(p['b1']),
        p['W2'], v(p['b2']), v(p['ln2h_g']), v(p['ln2h_b']))
    return (h, e)


# speedup vs baseline: 55.8395x; 1.1499x over previous
import functools

import jax
import jax.numpy as jnp
from jax import lax
from jax.experimental import pallas as pl
from jax.experimental.pallas import tpu as pltpu

N = 10000
E = 320000
HID = 128
H = 8
D = 16
CLAMP = 5.0

BN = 1000
BE = 1280


def _ln_rows(x, g, b):
    m = jnp.mean(x, axis=-1, keepdims=True)
    v = jnp.mean((x - m) ** 2, axis=-1, keepdims=True)
    return (x - m) * lax.rsqrt(v + 1e-05) * g + b


def _qkv_body(x_ref, wq_ref, wk_ref, wv_ref, q_ref, k_ref, v_ref):
    x = x_ref[...]
    q_ref[...] = jnp.dot(x, wq_ref[...], preferred_element_type=jnp.float32)
    k_ref[...] = jnp.dot(x, wk_ref[...], preferred_element_type=jnp.float32)
    v_ref[...] = jnp.dot(x, wv_ref[...], preferred_element_type=jnp.float32)


def _qkv(x, wq, wk, wv):
    full = pl.BlockSpec((HID, HID), lambda i: (0, 0))
    row = pl.BlockSpec((BN, HID), lambda i: (i, 0))
    return pl.pallas_call(
        _qkv_body,
        grid=(N // BN,),
        in_specs=[row, full, full, full],
        out_specs=[row, row, row],
        out_shape=[jax.ShapeDtypeStruct((N, HID), jnp.float32)] * 3,
    )(x, wq, wk, wv)


def _edge_body(pc_ref, qd_ref, ks_ref, vs_ref,
               wew_ref, web_ref, beb_ref, weo_ref, beo_ref,
               s128_ref, woe_ref, boe_ref, g_ref, b_ref,
               e_ref, valv_ref, valc_ref, p128_ref):
    pc = pc_ref[...]
    msg = qd_ref[...] + ks_ref[...]
    ew = jnp.dot(pc, wew_ref[...], preferred_element_type=jnp.float32)
    eb = jnp.dot(pc, web_ref[...], preferred_element_type=jnp.float32) + beb_ref[...]
    c1 = msg * ew
    c2 = jnp.sqrt(jnp.maximum(c1, 0.0)) - jnp.sqrt(jnp.maximum(-c1, 0.0))
    conn = jnp.maximum(c2 + eb, 0.0)
    C = jnp.dot(conn, weo_ref[...], preferred_element_type=jnp.float32) + beo_ref[...]
    score = jnp.clip(jnp.dot(C, s128_ref[...], preferred_element_type=jnp.float32),
                     -CLAMP, CLAMP)
    p128 = jnp.exp(score)
    valv_ref[...] = p128 * vs_ref[...]
    valc_ref[...] = p128 * C
    p128_ref[...] = p128
    e0 = pc + jnp.dot(C, woe_ref[...], preferred_element_type=jnp.float32) + boe_ref[...]
    e_ref[...] = _ln_rows(e0, g_ref[...], b_ref[...])


def _edge_pipeline(pc, qd, ks, vs, wew, web, beb, weo, beo, s128,
                   woe, boe, g, b):
    row = pl.BlockSpec((BE, HID), lambda i: (i, 0))
    full = pl.BlockSpec((HID, HID), lambda i: (0, 0))
    vec = pl.BlockSpec((1, HID), lambda i: (0, 0))
    return pl.pallas_call(
        _edge_body,
        grid=(E // BE,),
        in_specs=[row, row, row, row, full, full, vec, full, vec,
                  full, full, vec, vec, vec],
        out_specs=[row, row, row, row],
        out_shape=[jax.ShapeDtypeStruct((E, HID), jnp.float32)] * 4,
    )(pc, qd, ks, vs, wew, web, beb, weo, beo, s128, woe, boe, g, b)


def _node_body(x_ref, aggv_ref, rowv_ref, ss0_ref, ss1_ref, bd_ref,
               woh_ref, boh_ref, g1_ref, b1_ref, w1_ref, bb1_ref,
               w2_ref, bb2_ref, g2_ref, b2_ref, h_ref):
    inv = 1.0 / (ss0_ref[...] + ss1_ref[...] + 1e-16)
    on = aggv_ref[...] * inv + jnp.dot(
        rowv_ref[...] * inv, bd_ref[...], preferred_element_type=jnp.float32)
    h1 = x_ref[...] + jnp.dot(on, woh_ref[...],
                              preferred_element_type=jnp.float32) + boh_ref[...]
    h1 = _ln_rows(h1, g1_ref[...], b1_ref[...])
    t = jnp.maximum(jnp.dot(h1, w1_ref[...],
                            preferred_element_type=jnp.float32) + bb1_ref[...], 0.0)
    h2 = jnp.dot(t, w2_ref[...], preferred_element_type=jnp.float32) + bb2_ref[...]
    h_ref[...] = _ln_rows(h1 + h2, g2_ref[...], b2_ref[...])


def _node_epilogue(x, aggv, rowv, ssum0, ssum1, bd, woh, boh, g1, b1,
                   w1, bb1, w2, bb2, g2, b2):
    row = pl.BlockSpec((BN, HID), lambda i: (i, 0))
    full = pl.BlockSpec((HID, HID), lambda i: (0, 0))
    vec = pl.BlockSpec((1, HID), lambda i: (0, 0))
    return pl.pallas_call(
        _node_body,
        grid=(N // BN,),
        in_specs=[row, row, row, row, row, full, full, vec, vec, vec,
                  pl.BlockSpec((HID, 2 * HID), lambda i: (0, 0)),
                  pl.BlockSpec((1, 2 * HID), lambda i: (0, 0)),
                  pl.BlockSpec((2 * HID, HID), lambda i: (0, 0)),
                  vec, vec, vec],
        out_specs=row,
        out_shape=jax.ShapeDtypeStruct((N, HID), jnp.float32),
    )(x, aggv, rowv, ssum0, ssum1, bd, woh, boh, g1, b1, w1, bb1, w2, bb2, g2, b2)


from jax.experimental.pallas import tpu_sc as plsc

_NC, _NS = 2, 16
_NW = _NC * _NS
_CH = 80


def _gather_body(qh, kh, vh, dst3, src3, qd, ks, vs,
                 dstv, srcv, q0, k0, v0, q1, k1, v1, semA, semB):
    epw = E // _NW
    nch = epw // _CH
    wid = lax.axis_index("s") * _NC + lax.axis_index("c")
    base = wid * epw

    pltpu.sync_copy(dst3.at[wid], dstv)
    pltpu.sync_copy(src3.at[wid], srcv)

    def issue(i, qb, kb, vb, sem):
        pltpu.async_copy(qh.at[dstv.at[i]], qb, sem)
        pltpu.async_copy(kh.at[srcv.at[i]], kb, sem)
        pltpu.async_copy(vh.at[srcv.at[i]], vb, sem)

    def wait(i, qb, kb, vb, sem):
        pltpu.make_async_copy(qh.at[dstv.at[i]], qb, sem).wait()
        pltpu.make_async_copy(kh.at[srcv.at[i]], kb, sem).wait()
        pltpu.make_async_copy(vh.at[srcv.at[i]], vb, sem).wait()

    def writeback(i, qb, kb, vb):
        off = base + i * _CH
        pltpu.sync_copy(qb, qd.at[pl.ds(off, _CH)])
        pltpu.sync_copy(kb, ks.at[pl.ds(off, _CH)])
        pltpu.sync_copy(vb, vs.at[pl.ds(off, _CH)])

    issue(0, q0, k0, v0, semA)
    issue(1, q1, k1, v1, semB)

    def pair(j, carry):
        i0 = 2 * j
        i1 = i0 + 1
        wait(i0, q0, k0, v0, semA)
        writeback(i0, q0, k0, v0)
        issue(jnp.minimum(i0 + 2, nch - 1), q0, k0, v0, semA)
        wait(i1, q1, k1, v1, semB)
        writeback(i1, q1, k1, v1)
        issue(jnp.minimum(i1 + 2, nch - 1), q1, k1, v1, semB)
        return carry

    lax.fori_loop(0, nch // 2, pair, 0)
    wait(nch - 1, q0, k0, v0, semA)
    writeback(nch - 1, q0, k0, v0)
    wait(nch - 1, q1, k1, v1, semB)


def _gather_qkv(qh, kh, vh, dst, src):
    mesh = plsc.VectorSubcoreMesh(core_axis_name="c", subcore_axis_name="s",
                                  num_cores=_NC, num_subcores=_NS)
    fn = pl.kernel(
        _gather_body,
        out_type=[jax.ShapeDtypeStruct((E, HID), jnp.float32)] * 3,
        mesh=mesh,
        scratch_types=[
            pltpu.VMEM((E // _NW // _CH, _CH), jnp.int32),
            pltpu.VMEM((E // _NW // _CH, _CH), jnp.int32),
            pltpu.VMEM((_CH, HID), jnp.float32),
            pltpu.VMEM((_CH, HID), jnp.float32),
            pltpu.VMEM((_CH, HID), jnp.float32),
            pltpu.VMEM((_CH, HID), jnp.float32),
            pltpu.VMEM((_CH, HID), jnp.float32),
            pltpu.VMEM((_CH, HID), jnp.float32),
            pltpu.SemaphoreType.DMA,
            pltpu.SemaphoreType.DMA,
        ],
    )
    dst3 = dst.reshape(_NW, E // _NW // _CH, _CH)
    src3 = src.reshape(_NW, E // _NW // _CH, _CH)
    return fn(qh, kh, vh, dst3, src3)


def _scatter_body(valv, valc, p128, dst3b, zrow128, aggv_out, rowv_out,
                  ss0_out, ss1_out, accum, dstv, vbuf0, vbuf1, sem0, sem1):
    c = lax.axis_index("c")
    s = lax.axis_index("s")
    row0 = s * 624
    tail = (s == _NS - 1)

    def zero_own_rows():
        pltpu.sync_copy(zrow128, vbuf0)

        def zstripe(j, carry):
            pltpu.sync_copy(vbuf0.at[pl.ds(0, 48)],
                            accum.at[pl.ds(row0 + j * 48, 48)])
            return carry

        lax.fori_loop(0, 13, zstripe, 0)

        @pl.when(tail)
        def _():
            pltpu.sync_copy(vbuf0.at[pl.ds(0, 16)], accum.at[pl.ds(9984, 16)])

    def write_own_rows(out):
        def wstripe(j, carry):
            r = row0 + j * 48
            pltpu.sync_copy(accum.at[pl.ds(r, 48)], vbuf0.at[pl.ds(0, 48)])
            pltpu.sync_copy(vbuf0.at[pl.ds(0, 48)], out.at[pl.ds(r, 48)])
            return carry

        lax.fori_loop(0, 13, wstripe, 0)

        @pl.when(tail)
        def _():
            pltpu.sync_copy(accum.at[pl.ds(9984, 16)], vbuf0.at[pl.ds(0, 16)])
            pltpu.sync_copy(vbuf0.at[pl.ds(0, 16)], out.at[pl.ds(9984, 16)])

    def scatter_range(val, idxrows, edge0, nstep):
        pltpu.sync_copy(idxrows, dstv.at[pl.ds(0, nstep)])

        def pair(j, carry):
            i0 = 2 * j
            i1 = i0 + 1

            @pl.when(j > 0)
            def _():
                pltpu.make_async_copy(
                    vbuf0, accum.at[dstv.at[i0 - 2]], sem0).wait()

            pltpu.sync_copy(val.at[pl.ds(edge0 + i0 * _CH, _CH)], vbuf0)
            pltpu.async_copy(vbuf0, accum.at[dstv.at[i0]], sem0, add=True)

            @pl.when(j > 0)
            def _():
                pltpu.make_async_copy(
                    vbuf1, accum.at[dstv.at[i1 - 2]], sem1).wait()

            pltpu.sync_copy(val.at[pl.ds(edge0 + i1 * _CH, _CH)], vbuf1)
            pltpu.async_copy(vbuf1, accum.at[dstv.at[i1]], sem1, add=True)
            return carry

        npair = nstep // 2
        lax.fori_loop(0, npair, pair, 0)
        pltpu.make_async_copy(
            vbuf0, accum.at[dstv.at[2 * npair - 2]], sem0).wait()
        pltpu.make_async_copy(
            vbuf1, accum.at[dstv.at[2 * npair - 1]], sem1).wait()
        if nstep % 2:
            i = nstep - 1
            pltpu.sync_copy(val.at[pl.ds(edge0 + i * _CH, _CH)], vbuf0)
            pltpu.sync_copy(vbuf0, accum.at[dstv.at[i]], add=True)

    zero_own_rows()
    plsc.subcore_barrier()
    cpw = (E // _NW) // _CH

    @pl.when(c == 0)
    def _():
        scatter_range(valv, dst3b.at[2 * s], 2 * s * cpw * _CH, cpw)
        scatter_range(valv, dst3b.at[2 * s + 1], (2 * s + 1) * cpw * _CH, cpw)

    @pl.when(c == 1)
    def _():
        scatter_range(valc, dst3b.at[2 * s], 2 * s * cpw * _CH, cpw)
        scatter_range(valc, dst3b.at[2 * s + 1], (2 * s + 1) * cpw * _CH, cpw)

    plsc.subcore_barrier()

    @pl.when(c == 0)
    def _():
        write_own_rows(aggv_out)

    @pl.when(c == 1)
    def _():
        write_own_rows(rowv_out)

    zero_own_rows()
    plsc.subcore_barrier()
    w = c * _NS + s
    scatter_range(p128, dst3b.at[w], w * cpw * _CH, cpw)
    plsc.subcore_barrier()

    @pl.when(c == 0)
    def _():
        write_own_rows(ss0_out)

    @pl.when(c == 1)
    def _():
        write_own_rows(ss1_out)


def _scatter_segments(valv, valc, p128, dst):
    mesh = plsc.VectorSubcoreMesh(core_axis_name="c", subcore_axis_name="s",
                                  num_cores=_NC, num_subcores=_NS)
    fn = pl.kernel(
        _scatter_body,
        out_type=[jax.ShapeDtypeStruct((N, HID), jnp.float32)] * 4,
        mesh=mesh,
        scratch_types=[
            pltpu.VMEM_SHARED((N, HID), jnp.float32),
            pltpu.VMEM((E // _NW // _CH, _CH), jnp.int32),
            pltpu.VMEM((_CH, HID), jnp.float32),
            pltpu.VMEM((_CH, HID), jnp.float32),
            pltpu.SemaphoreType.DMA,
            pltpu.SemaphoreType.DMA,
        ],
    )
    zrow128 = jnp.zeros((_CH, HID), jnp.float32)
    dst3b = dst.reshape(_NW, E // _NW // _CH, _CH)
    return fn(valv, valc, p128, dst3b, zrow128)


def kernel(x, poly_index, poly_conn, params):
    p = params
    dst = poly_index[0]
    src = poly_index[1]

    aw = p['Aw'][:, :, 0]
    lane = jnp.arange(HID)
    head = lane // D
    s128 = jax.scipy.linalg.block_diag(
        *[jnp.outer(aw[:, h], jnp.ones((D,), jnp.float32)) for h in range(H)])
    bd = jax.scipy.linalg.block_diag(*[p['BW'][:, h, :] for h in range(H)])

    def v(a):
        return a.reshape(1, -1)

    qh, kh, vh = _qkv(x, p['WQ'], p['WK'], p['WV'])
    qd, ks, vs = _gather_qkv(qh, kh, vh, dst, src)
    e, valv, valc, p128 = _edge_pipeline(
        poly_conn, qd, ks, vs, p['WEw'], p['WEb'], v(p['bEb']), p['WEo'],
        v(p['bEo']), s128, p['WOe'], v(p['bOe']),
        v(p['ln1e_g']), v(p['ln1e_b']))
    aggv, rowv, ss0, ss1 = _scatter_segments(valv, valc, p128, dst)
    h = _node_epilogue(
        x, aggv, rowv, ss0, ss1, bd, p['WOh'], v(p['bOh']),
        v(p['ln1h_g']), v(p['ln1h_b']), p['W1'], v(p['b1']),
        p['W2'], v(p['b2']), v(p['ln2h_g']), v(p['ln2h_b']))
    return (h, e)
